# Initial kernel scaffold; baseline (speedup 1.0000x reference)
import jax
import jax.numpy as jnp
from jax.experimental import pallas as pl


def kernel(x, edge_attr, frag_ea, cond, params, edge_index, atom_to_frag, frag_ei, frag_batch):
    raise NotImplementedError("write your pallas kernel here")


# speedup vs baseline: 2.3988x; 2.3988x over previous
import functools

import jax
import jax.numpy as jnp
from jax import lax
from jax.experimental import pallas as pl
from jax.experimental.pallas import tpu as pltpu
from jax.experimental.pallas import tpu_sc as plsc


def _linear_body(x_ref, w_ref, b_ref, o_ref):
    o_ref[...] = (
        jnp.dot(x_ref[...], w_ref[...], preferred_element_type=jnp.float32)
        + b_ref[...]
    )


def _linear(x, w, b, rb):
    n, din = x.shape
    dout = w.shape[1]
    return pl.pallas_call(
        _linear_body,
        grid=(n // rb,),
        in_specs=[
            pl.BlockSpec((rb, din), lambda i: (i, 0)),
            pl.BlockSpec((din, dout), lambda i: (0, 0)),
            pl.BlockSpec((1, dout), lambda i: (0, 0)),
        ],
        out_specs=pl.BlockSpec((rb, dout), lambda i: (i, 0)),
        out_shape=jax.ShapeDtypeStruct((n, dout), jnp.float32),
    )(x, w, b.reshape(1, -1))


def _edge_embed_body(ea_ref, w_ref, b_ref, o_ref):
    o_ref[0] = (
        jnp.dot(ea_ref[...], w_ref[0], preferred_element_type=jnp.float32)
        + b_ref[0]
    )


def _edge_embed(ea, w, b, e_pad, eb):
    L, de, h = w.shape
    return pl.pallas_call(
        _edge_embed_body,
        grid=(L, e_pad // eb),
        in_specs=[
            pl.BlockSpec((eb, de), lambda l, i: (i, 0)),
            pl.BlockSpec((1, de, h), lambda l, i: (l, 0, 0)),
            pl.BlockSpec((1, 1, h), lambda l, i: (l, 0, 0)),
        ],
        out_specs=pl.BlockSpec((1, eb, h), lambda l, i: (l, i, 0)),
        out_shape=jax.ShapeDtypeStruct((L, e_pad, h), jnp.float32),
    )(ea, w, b.reshape(L, 1, h))


def _gine_mlp_body(eps_ref, x_ref, agg_ref, w1_ref, b1_ref, w2_ref, b2_ref,
                   g_ref, bb_ref, o_ref):
    eps = eps_ref[0]
    dd = x_ref.shape[1]
    agg = agg_ref[0][:, :dd] + agg_ref[1][:, :dd]
    h = (1.0 + eps) * x_ref[...] + agg
    t = jnp.maximum(
        jnp.dot(h, w1_ref[...], preferred_element_type=jnp.float32)
        + b1_ref[...], 0.0)
    o = (jnp.dot(t, w2_ref[...], preferred_element_type=jnp.float32)
         + b2_ref[...])
    mu = jnp.mean(o, axis=-1, keepdims=True)
    var = jnp.mean(o * o, axis=-1, keepdims=True) - mu * mu
    o = (o - mu) * lax.rsqrt(var + 1e-5) * g_ref[...] + bb_ref[...]
    o_ref[...] = jnp.maximum(o, 0.0)


def _gine_mlp(xx, agg2, eps, w1, b1, w2, b2, g, bb, rb):
    n, dd = xx.shape
    da = agg2.shape[2]
    d2 = w1.shape[1]
    return pl.pallas_call(
        _gine_mlp_body,
        grid=(n // rb,),
        in_specs=[
            pl.BlockSpec(memory_space=pltpu.SMEM),
            pl.BlockSpec((rb, dd), lambda i: (i, 0)),
            pl.BlockSpec((2, rb, da), lambda i: (0, i, 0)),
            pl.BlockSpec((dd, d2), lambda i: (0, 0)),
            pl.BlockSpec((1, d2), lambda i: (0, 0)),
            pl.BlockSpec((d2, dd), lambda i: (0, 0)),
            pl.BlockSpec((1, dd), lambda i: (0, 0)),
            pl.BlockSpec((1, dd), lambda i: (0, 0)),
            pl.BlockSpec((1, dd), lambda i: (0, 0)),
        ],
        out_specs=pl.BlockSpec((rb, dd), lambda i: (i, 0)),
        out_shape=jax.ShapeDtypeStruct((n, dd), jnp.float32),
    )(eps.reshape(1), xx, agg2, w1, b1.reshape(1, -1), w2,
      b2.reshape(1, -1), g.reshape(1, -1), bb.reshape(1, -1))


def _pool_body(a2f_ref, h_ref, w1_ref, b1_ref, w2_ref, b2_ref, o_ref):
    fb, _ = o_ref.shape
    nfull = h_ref.shape[0]
    f0 = pl.program_id(0) * fb
    fid = lax.broadcasted_iota(jnp.int32, (fb, nfull), 0) + f0
    oh = (fid == a2f_ref[...]).astype(jnp.float32)
    sums = jnp.dot(oh, h_ref[...], preferred_element_type=jnp.float32)
    cnt = jnp.sum(oh, axis=1, keepdims=True)
    m = sums / jnp.maximum(cnt, 1.0)
    t = (jnp.dot(m, w1_ref[...], preferred_element_type=jnp.float32)
         + b1_ref[...])
    o_ref[...] = (jnp.dot(t, w2_ref[...], preferred_element_type=jnp.float32)
                  + b2_ref[...])


def _pool(a2f, h_atom, w1, b1, w2, b2, nf, fb):
    n, dd = h_atom.shape
    hf = w1.shape[1]
    return pl.pallas_call(
        _pool_body,
        grid=(nf // fb,),
        in_specs=[
            pl.BlockSpec((1, n), lambda i: (0, 0)),
            pl.BlockSpec((n, dd), lambda i: (0, 0)),
            pl.BlockSpec((dd, hf), lambda i: (0, 0)),
            pl.BlockSpec((1, hf), lambda i: (0, 0)),
            pl.BlockSpec((hf, hf), lambda i: (0, 0)),
            pl.BlockSpec((1, hf), lambda i: (0, 0)),
        ],
        out_specs=pl.BlockSpec((fb, hf), lambda i: (i, 0)),
        out_shape=jax.ShapeDtypeStruct((nf, hf), jnp.float32),
    )(a2f.reshape(1, -1), h_atom, w1, b1.reshape(1, -1), w2,
      b2.reshape(1, -1))


def _final_body(hf_ref, fb_ref, cond_ref, fow1_ref, fob1_ref, fow2_ref,
                fob2_ref, cpw1_ref, cpb1_ref, cpw2_ref, cpb2_ref, gmw1_ref,
                gmb1_ref, gmw2_ref, gmb2_ref, dg_ref, dl_ref):
    hf = hf_ref[...]
    nb = dg_ref.shape[0]
    nf = hf.shape[0]
    hm = hf_ref.shape[1]
    t = jnp.maximum(
        jnp.dot(hf, fow1_ref[...], preferred_element_type=jnp.float32)
        + fob1_ref[...], 0.0)
    deltas = (jnp.dot(t, fow2_ref[...], preferred_element_type=jnp.float32)
              + fob2_ref[...])
    dl_ref[...] = deltas
    oh = (lax.broadcasted_iota(jnp.int32, (nb, nf), 0)
          == fb_ref[...]).astype(jnp.float32)
    fs = jnp.dot(oh, deltas, preferred_element_type=jnp.float32)
    cnt = jnp.sum(oh, axis=1, keepdims=True)
    hmol = (jnp.dot(oh, hf, preferred_element_type=jnp.float32)
            / jnp.maximum(cnt, 1.0))
    hc = jnp.maximum(
        jnp.dot(cond_ref[...], cpw1_ref[...],
                preferred_element_type=jnp.float32) + cpb1_ref[...], 0.0)
    hc = (jnp.dot(hc, cpw2_ref[...], preferred_element_type=jnp.float32)
          + cpb2_ref[...])
    gw = gmw1_ref[...]
    z = (jnp.dot(hmol, gw[:hm, :], preferred_element_type=jnp.float32)
         + jnp.dot(hc, gw[hm:, :], preferred_element_type=jnp.float32)
         + gmb1_ref[...])
    g = (jnp.dot(jnp.maximum(z, 0.0), gmw2_ref[...],
                 preferred_element_type=jnp.float32) + gmb2_ref[...])
    dg_ref[...] = fs + g


def _final(h_frag, frag_batch, cond, p):
    nf, hm = h_frag.shape
    nb = cond.shape[0]
    return pl.pallas_call(
        _final_body,
        out_shape=[
            jax.ShapeDtypeStruct((nb, 1), jnp.float32),
            jax.ShapeDtypeStruct((nf, 1), jnp.float32),
        ],
    )(h_frag, frag_batch.reshape(1, -1), cond,
      p['fo_W1'], p['fo_b1'].reshape(1, -1),
      p['fo_W2'], p['fo_b2'].reshape(1, -1),
      p['cp_W1'], p['cp_b1'].reshape(1, -1),
      p['cp_W2'], p['cp_b2'].reshape(1, -1),
      p['gm_W1'], p['gm_b1'].reshape(1, -1),
      p['gm_W2'], p['gm_b2'].reshape(1, -1))


_CH = 128
_ZR = 32


def _sc_edge_pass(x, ea3, src, dst, l, e_pad, n_acc):
    d = x.shape[1]
    epw = e_pad // 32
    chunks = epw // _CH
    rpt = n_acc // 16
    zcopies = rpt // _ZR
    dpe = d // 16

    mesh = plsc.VectorSubcoreMesh(core_axis_name="c", subcore_axis_name="s")

    @functools.partial(
        pl.kernel,
        mesh=mesh,
        out_type=jax.ShapeDtypeStruct((2, n_acc, d), jnp.float32),
        scratch_types=[
            pltpu.VMEM((_CH,), jnp.int32),
            pltpu.VMEM((_CH,), jnp.int32),
            pltpu.VMEM((_CH, d), jnp.float32),
            pltpu.VMEM((_CH, d), jnp.float32),
            pltpu.VMEM((_ZR, d), jnp.float32),
            pltpu.VMEM_SHARED((n_acc, d), jnp.float32),
            pltpu.SemaphoreType.DMA,
        ],
    )
    def k(x_hbm, ea_hbm, src_hbm, dst_hbm, out_hbm,
          sidx, didx, rows, msg, zbuf, acc, sem):
        c = lax.axis_index("c")
        s = lax.axis_index("s")

        def zrow(r, carry):
            for j in range(dpe):
                zbuf[r, pl.ds(j * 16, 16)] = jnp.zeros((16,), jnp.float32)
            return carry
        lax.fori_loop(0, _ZR, zrow, 0)

        def zcp(kk, carry):
            pltpu.sync_copy(zbuf, acc.at[pl.ds(s * rpt + kk * _ZR, _ZR)])
            return carry
        lax.fori_loop(0, zcopies, zcp, 0)
        plsc.subcore_barrier()

        tile_base = c * (e_pad // 2) + s * epw

        def chunk(g, carry):
            off = tile_base + g * _CH
            pltpu.sync_copy(src_hbm.at[pl.ds(off, _CH)], sidx)
            pltpu.sync_copy(dst_hbm.at[pl.ds(off, _CH)], didx)
            cp1 = pltpu.async_copy(ea_hbm.at[l, pl.ds(off, _CH)], msg, sem)
            cp2 = pltpu.async_copy(x_hbm.at[sidx], rows, sem)
            cp1.wait()
            cp2.wait()

            def rrow(r, cc):
                for j in range(dpe):
                    sl = pl.ds(j * 16, 16)
                    msg[r, sl] = jnp.maximum(msg[r, sl] + rows[r, sl], 0.0)
                return cc
            lax.fori_loop(0, _CH, rrow, 0)
            pltpu.sync_copy(msg, acc.at[didx], add=True)
            return carry
        lax.fori_loop(0, chunks, chunk, 0)
        plsc.subcore_barrier()
        pltpu.sync_copy(acc.at[pl.ds(s * rpt, rpt)],
                        out_hbm.at[c, pl.ds(s * rpt, rpt)])

    return k(x, ea3, src, dst)


def _ceil_to(v, m):
    return -(-v // m) * m


def kernel(x, edge_attr, frag_ea, cond, params, edge_index, atom_to_frag,
           frag_ei, frag_batch):
    pa = params['atom']
    pf = params['frag']
    N, _ = x.shape
    E = edge_attr.shape[0]
    H = pa['proj_W'].shape[1]
    F = frag_batch.shape[0]
    EF = frag_ea.shape[0]
    HF = pf['proj_W'].shape[1]
    LA = pa['edge_W'].shape[0]
    LF = pf['edge_W'].shape[0]

    e_pad = _ceil_to(E, 32 * _CH)
    n_acc = _ceil_to(N + 1, 16 * _ZR)
    ef_pad = _ceil_to(EF, 32 * _CH)
    f_acc = _ceil_to(F + 1, 16 * _ZR)

    srcA = jnp.concatenate(
        [edge_index[0].astype(jnp.int32),
         jnp.zeros((e_pad - E,), jnp.int32)])
    dstA = jnp.concatenate(
        [edge_index[1].astype(jnp.int32),
         jnp.full((e_pad - E,), N, jnp.int32)])
    srcF = jnp.concatenate(
        [frag_ei[0].astype(jnp.int32),
         jnp.zeros((ef_pad - EF,), jnp.int32)])
    dstF = jnp.concatenate(
        [frag_ei[1].astype(jnp.int32),
         jnp.full((ef_pad - EF,), F, jnp.int32)])

    ea3 = _edge_embed(edge_attr, pa['edge_W'], pa['edge_b'], e_pad, 4096)
    h = _linear(x, pa['proj_W'], pa['proj_b'], 1000)
    for l in range(LA):
        agg2 = _sc_edge_pass(h, ea3, srcA, dstA, l, e_pad, n_acc)
        h = _gine_mlp(h, agg2, pa['eps'][l], pa['mlp_W1'][l], pa['mlp_b1'][l],
                      pa['mlp_W2'][l], pa['mlp_b2'][l], pa['ln_g'][l],
                      pa['ln_b'][l], 1000)

    hw = _pool(atom_to_frag.astype(jnp.int32), h,
               params['frag_proj_W'], params['frag_proj_b'],
               pf['proj_W'], pf['proj_b'], F, 400)

    ew_pad = jnp.pad(pf['edge_W'], ((0, 0), (0, 0), (0, H - HF)))
    eb_pad = jnp.pad(pf['edge_b'], ((0, 0), (0, H - HF)))
    eaf = _edge_embed(frag_ea, ew_pad, eb_pad, ef_pad, 4096)
    for l in range(LF):
        hwp = jnp.pad(hw, ((0, 0), (0, H - HF)))
        agg2 = _sc_edge_pass(hwp, eaf, srcF, dstF, l, ef_pad, f_acc)
        hw = _gine_mlp(hw, agg2, pf['eps'][l], pf['mlp_W1'][l],
                       pf['mlp_b1'][l], pf['mlp_W2'][l], pf['mlp_b2'][l],
                       pf['ln_g'][l], pf['ln_b'][l], 1000)

    dg, deltas = _final(hw, frag_batch.astype(jnp.int32), cond, params)
    return dg, deltas.reshape(-1)


# speedup vs baseline: 2.9121x; 1.2140x over previous
import functools

import jax
import jax.numpy as jnp
from jax import lax
from jax.experimental import pallas as pl
from jax.experimental.pallas import tpu as pltpu
from jax.experimental.pallas import tpu_sc as plsc


def _linear_body(x_ref, w_ref, b_ref, o_ref):
    o_ref[...] = (
        jnp.dot(x_ref[...], w_ref[...], preferred_element_type=jnp.float32)
        + b_ref[...]
    )


def _linear(x, w, b, rb):
    n, din = x.shape
    dout = w.shape[1]
    return pl.pallas_call(
        _linear_body,
        grid=(n // rb,),
        in_specs=[
            pl.BlockSpec((rb, din), lambda i: (i, 0)),
            pl.BlockSpec((din, dout), lambda i: (0, 0)),
            pl.BlockSpec((1, dout), lambda i: (0, 0)),
        ],
        out_specs=pl.BlockSpec((rb, dout), lambda i: (i, 0)),
        out_shape=jax.ShapeDtypeStruct((n, dout), jnp.float32),
    )(x, w, b.reshape(1, -1))


def _edge_embed_body(ea_ref, w_ref, b_ref, o_ref):
    o_ref[0] = (
        jnp.dot(ea_ref[...], w_ref[0], preferred_element_type=jnp.float32)
        + b_ref[0]
    )


def _edge_embed(ea, w, b, e_pad, eb):
    L, de, h = w.shape
    return pl.pallas_call(
        _edge_embed_body,
        grid=(L, e_pad // eb),
        in_specs=[
            pl.BlockSpec((eb, de), lambda l, i: (i, 0)),
            pl.BlockSpec((1, de, h), lambda l, i: (l, 0, 0)),
            pl.BlockSpec((1, 1, h), lambda l, i: (l, 0, 0)),
        ],
        out_specs=pl.BlockSpec((1, eb, h), lambda l, i: (l, i, 0)),
        out_shape=jax.ShapeDtypeStruct((L, e_pad, h), jnp.float32),
    )(ea, w, b.reshape(L, 1, h))


def _gine_mlp_body(eps_ref, x_ref, agg_ref, w1_ref, b1_ref, w2_ref, b2_ref,
                   g_ref, bb_ref, o_ref):
    eps = eps_ref[0]
    dd = x_ref.shape[1]
    agg = agg_ref[0][:, :dd] + agg_ref[1][:, :dd]
    h = (1.0 + eps) * x_ref[...] + agg
    t = jnp.maximum(
        jnp.dot(h, w1_ref[...], preferred_element_type=jnp.float32)
        + b1_ref[...], 0.0)
    o = (jnp.dot(t, w2_ref[...], preferred_element_type=jnp.float32)
         + b2_ref[...])
    mu = jnp.mean(o, axis=-1, keepdims=True)
    var = jnp.mean(o * o, axis=-1, keepdims=True) - mu * mu
    o = (o - mu) * lax.rsqrt(var + 1e-5) * g_ref[...] + bb_ref[...]
    o_ref[...] = jnp.maximum(o, 0.0)


def _gine_mlp(xx, agg2, eps, w1, b1, w2, b2, g, bb, rb):
    n, dd = xx.shape
    da = agg2.shape[2]
    d2 = w1.shape[1]
    return pl.pallas_call(
        _gine_mlp_body,
        grid=(n // rb,),
        in_specs=[
            pl.BlockSpec(memory_space=pltpu.SMEM),
            pl.BlockSpec((rb, dd), lambda i: (i, 0)),
            pl.BlockSpec((2, rb, da), lambda i: (0, i, 0)),
            pl.BlockSpec((dd, d2), lambda i: (0, 0)),
            pl.BlockSpec((1, d2), lambda i: (0, 0)),
            pl.BlockSpec((d2, dd), lambda i: (0, 0)),
            pl.BlockSpec((1, dd), lambda i: (0, 0)),
            pl.BlockSpec((1, dd), lambda i: (0, 0)),
            pl.BlockSpec((1, dd), lambda i: (0, 0)),
        ],
        out_specs=pl.BlockSpec((rb, dd), lambda i: (i, 0)),
        out_shape=jax.ShapeDtypeStruct((n, dd), jnp.float32),
    )(eps.reshape(1), xx, agg2, w1, b1.reshape(1, -1), w2,
      b2.reshape(1, -1), g.reshape(1, -1), bb.reshape(1, -1))


def _pool_body(a2f_ref, h_ref, w1_ref, b1_ref, w2_ref, b2_ref, o_ref):
    fb, _ = o_ref.shape
    nfull = h_ref.shape[0]
    f0 = pl.program_id(0) * fb
    fid = lax.broadcasted_iota(jnp.int32, (fb, nfull), 0) + f0
    oh = (fid == a2f_ref[...]).astype(jnp.float32)
    sums = jnp.dot(oh, h_ref[...], preferred_element_type=jnp.float32)
    cnt = jnp.sum(oh, axis=1, keepdims=True)
    m = sums / jnp.maximum(cnt, 1.0)
    t = (jnp.dot(m, w1_ref[...], preferred_element_type=jnp.float32)
         + b1_ref[...])
    o_ref[...] = (jnp.dot(t, w2_ref[...], preferred_element_type=jnp.float32)
                  + b2_ref[...])


def _pool(a2f, h_atom, w1, b1, w2, b2, nf, fb):
    n, dd = h_atom.shape
    hf = w1.shape[1]
    return pl.pallas_call(
        _pool_body,
        grid=(nf // fb,),
        in_specs=[
            pl.BlockSpec((1, n), lambda i: (0, 0)),
            pl.BlockSpec((n, dd), lambda i: (0, 0)),
            pl.BlockSpec((dd, hf), lambda i: (0, 0)),
            pl.BlockSpec((1, hf), lambda i: (0, 0)),
            pl.BlockSpec((hf, hf), lambda i: (0, 0)),
            pl.BlockSpec((1, hf), lambda i: (0, 0)),
        ],
        out_specs=pl.BlockSpec((fb, hf), lambda i: (i, 0)),
        out_shape=jax.ShapeDtypeStruct((nf, hf), jnp.float32),
    )(a2f.reshape(1, -1), h_atom, w1, b1.reshape(1, -1), w2,
      b2.reshape(1, -1))


def _final_body(hf_ref, fb_ref, cond_ref, fow1_ref, fob1_ref, fow2_ref,
                fob2_ref, cpw1_ref, cpb1_ref, cpw2_ref, cpb2_ref, gmw1_ref,
                gmb1_ref, gmw2_ref, gmb2_ref, dg_ref, dl_ref):
    hf = hf_ref[...]
    nb = dg_ref.shape[0]
    nf = hf.shape[0]
    hm = hf_ref.shape[1]
    t = jnp.maximum(
        jnp.dot(hf, fow1_ref[...], preferred_element_type=jnp.float32)
        + fob1_ref[...], 0.0)
    deltas = (jnp.dot(t, fow2_ref[...], preferred_element_type=jnp.float32)
              + fob2_ref[...])
    dl_ref[...] = deltas
    oh = (lax.broadcasted_iota(jnp.int32, (nb, nf), 0)
          == fb_ref[...]).astype(jnp.float32)
    fs = jnp.dot(oh, deltas, preferred_element_type=jnp.float32)
    cnt = jnp.sum(oh, axis=1, keepdims=True)
    hmol = (jnp.dot(oh, hf, preferred_element_type=jnp.float32)
            / jnp.maximum(cnt, 1.0))
    hc = jnp.maximum(
        jnp.dot(cond_ref[...], cpw1_ref[...],
                preferred_element_type=jnp.float32) + cpb1_ref[...], 0.0)
    hc = (jnp.dot(hc, cpw2_ref[...], preferred_element_type=jnp.float32)
          + cpb2_ref[...])
    gw = gmw1_ref[...]
    z = (jnp.dot(hmol, gw[:hm, :], preferred_element_type=jnp.float32)
         + jnp.dot(hc, gw[hm:, :], preferred_element_type=jnp.float32)
         + gmb1_ref[...])
    g = (jnp.dot(jnp.maximum(z, 0.0), gmw2_ref[...],
                 preferred_element_type=jnp.float32) + gmb2_ref[...])
    dg_ref[...] = fs + g


def _final(h_frag, frag_batch, cond, p):
    nf, hm = h_frag.shape
    nb = cond.shape[0]
    return pl.pallas_call(
        _final_body,
        out_shape=[
            jax.ShapeDtypeStruct((nb, 1), jnp.float32),
            jax.ShapeDtypeStruct((nf, 1), jnp.float32),
        ],
    )(h_frag, frag_batch.reshape(1, -1), cond,
      p['fo_W1'], p['fo_b1'].reshape(1, -1),
      p['fo_W2'], p['fo_b2'].reshape(1, -1),
      p['cp_W1'], p['cp_b1'].reshape(1, -1),
      p['cp_W2'], p['cp_b2'].reshape(1, -1),
      p['gm_W1'], p['gm_b1'].reshape(1, -1),
      p['gm_W2'], p['gm_b2'].reshape(1, -1))


_ZR = 8


def _sc_edge_pass(x, ea3, src, dst, l, e_pad, n_acc, ch):
    d = x.shape[1]
    epw = e_pad // 32
    chunks = epw // ch
    rpt = n_acc // 16
    zcopies = rpt // _ZR
    dpe = d // 16

    mesh = plsc.VectorSubcoreMesh(core_axis_name="c", subcore_axis_name="s")

    @functools.partial(
        pl.kernel,
        mesh=mesh,
        out_type=jax.ShapeDtypeStruct((2, n_acc, d), jnp.float32),
        scratch_types=[
            pltpu.VMEM((ch,), jnp.int32),
            pltpu.VMEM((ch,), jnp.int32),
            pltpu.VMEM((ch, d), jnp.float32),
            pltpu.VMEM((ch, d), jnp.float32),
            pltpu.VMEM((ch,), jnp.int32),
            pltpu.VMEM((ch,), jnp.int32),
            pltpu.VMEM((ch, d), jnp.float32),
            pltpu.VMEM((ch, d), jnp.float32),
            pltpu.VMEM((_ZR, d), jnp.float32),
            pltpu.VMEM_SHARED((n_acc, d), jnp.float32),
            pltpu.SemaphoreType.DMA,
            pltpu.SemaphoreType.DMA,
            pltpu.SemaphoreType.DMA,
            pltpu.SemaphoreType.DMA,
        ],
    )
    def k(x_hbm, ea_hbm, src_hbm, dst_hbm, out_hbm,
          sidxA, didxA, rowsA, msgA, sidxB, didxB, rowsB, msgB,
          zbuf, acc, semA, semB, ssemA, ssemB):
        c = lax.axis_index("c")
        s = lax.axis_index("s")

        def zrow(r, carry):
            for j in range(dpe):
                zbuf[r, pl.ds(j * 16, 16)] = jnp.zeros((16,), jnp.float32)
            return carry
        lax.fori_loop(0, _ZR, zrow, 0)

        def zcp(kk, carry):
            pltpu.sync_copy(zbuf, acc.at[pl.ds(s * rpt + kk * _ZR, _ZR)])
            return carry
        lax.fori_loop(0, zcopies, zcp, 0)
        plsc.subcore_barrier()

        tile_base = c * (e_pad // 2) + s * epw
        bufA = (sidxA, didxA, rowsA, msgA, semA, ssemA)
        bufB = (sidxB, didxB, rowsB, msgB, semB, ssemB)

        def load_and_issue(g, bufs):
            sidx, didx, rows, msg, sem, _ = bufs
            off = tile_base + g * ch
            pltpu.sync_copy(src_hbm.at[pl.ds(off, ch)], sidx)
            pltpu.sync_copy(dst_hbm.at[pl.ds(off, ch)], didx)
            pltpu.async_copy(ea_hbm.at[l, pl.ds(off, ch)], msg, sem)
            pltpu.async_copy(x_hbm.at[sidx], rows, sem)

        def process(g, bufs, obufs):
            sidx, didx, rows, msg, sem, ssem = bufs
            _, odidx, _, omsg, _, ossem = obufs
            off = tile_base + g * ch
            pltpu.make_async_copy(ea_hbm.at[l, pl.ds(off, ch)], msg,
                                  sem).wait()
            pltpu.make_async_copy(x_hbm.at[sidx], rows, sem).wait()

            def rrow(r, cc):
                for j in range(dpe):
                    sl = pl.ds(j * 16, 16)
                    msg[r, sl] = jnp.maximum(msg[r, sl] + rows[r, sl], 0.0)
                return cc
            lax.fori_loop(0, ch, rrow, 0)

            @pl.when(g >= 1)
            def _():
                pltpu.make_async_copy(omsg, acc.at[odidx], ossem).wait()
            pltpu.async_copy(msg, acc.at[didx], ssem, add=True)

            @pl.when(g + 1 < chunks)
            def _():
                load_and_issue(g + 1, obufs)

        load_and_issue(0, bufA)

        def pair(p, carry):
            process(2 * p, bufA, bufB)
            process(2 * p + 1, bufB, bufA)
            return carry
        lax.fori_loop(0, chunks // 2, pair, 0)
        if chunks % 2:
            process(chunks - 1, bufA, bufB)

        lmsg, ldidx, lssem = ((msgA, didxA, ssemA) if chunks % 2
                              else (msgB, didxB, ssemB))
        pltpu.make_async_copy(lmsg, acc.at[ldidx], lssem).wait()
        plsc.subcore_barrier()
        pltpu.sync_copy(acc.at[pl.ds(s * rpt, rpt)],
                        out_hbm.at[c, pl.ds(s * rpt, rpt)])

    return k(x, ea3, src, dst)


def _ceil_to(v, m):
    return -(-v // m) * m


def kernel(x, edge_attr, frag_ea, cond, params, edge_index, atom_to_frag,
           frag_ei, frag_batch):
    pa = params['atom']
    pf = params['frag']
    N, _ = x.shape
    E = edge_attr.shape[0]
    H = pa['proj_W'].shape[1]
    F = frag_batch.shape[0]
    EF = frag_ea.shape[0]
    HF = pf['proj_W'].shape[1]
    LA = pa['edge_W'].shape[0]
    LF = pf['edge_W'].shape[0]

    ch_a = 80
    ch_f = 64
    e_pad = _ceil_to(E, 32 * ch_a)
    n_acc = _ceil_to(N + 1, 16 * _ZR)
    ef_pad = _ceil_to(EF, 32 * ch_f)
    f_acc = _ceil_to(F + 1, 16 * _ZR)

    srcA = jnp.concatenate(
        [edge_index[0].astype(jnp.int32),
         jnp.zeros((e_pad - E,), jnp.int32)])
    dstA = jnp.concatenate(
        [edge_index[1].astype(jnp.int32),
         jnp.full((e_pad - E,), N, jnp.int32)])
    srcF = jnp.concatenate(
        [frag_ei[0].astype(jnp.int32),
         jnp.zeros((ef_pad - EF,), jnp.int32)])
    dstF = jnp.concatenate(
        [frag_ei[1].astype(jnp.int32),
         jnp.full((ef_pad - EF,), F, jnp.int32)])

    ea3 = _edge_embed(edge_attr, pa['edge_W'], pa['edge_b'], e_pad, 4000)
    h = _linear(x, pa['proj_W'], pa['proj_b'], 1000)
    for l in range(LA):
        agg2 = _sc_edge_pass(h, ea3, srcA, dstA, l, e_pad, n_acc, ch_a)
        h = _gine_mlp(h, agg2, pa['eps'][l], pa['mlp_W1'][l], pa['mlp_b1'][l],
                      pa['mlp_W2'][l], pa['mlp_b2'][l], pa['ln_g'][l],
                      pa['ln_b'][l], 1000)

    hw = _pool(atom_to_frag.astype(jnp.int32), h,
               params['frag_proj_W'], params['frag_proj_b'],
               pf['proj_W'], pf['proj_b'], F, 400)

    ew_pad = jnp.pad(pf['edge_W'], ((0, 0), (0, 0), (0, H - HF)))
    eb_pad = jnp.pad(pf['edge_b'], ((0, 0), (0, H - HF)))
    eaf = _edge_embed(frag_ea, ew_pad, eb_pad, ef_pad, 4096)
    for l in range(LF):
        hwp = jnp.pad(hw, ((0, 0), (0, H - HF)))
        agg2 = _sc_edge_pass(hwp, eaf, srcF, dstF, l, ef_pad, f_acc, ch_f)
        hw = _gine_mlp(hw, agg2, pf['eps'][l], pf['mlp_W1'][l],
                       pf['mlp_b1'][l], pf['mlp_W2'][l], pf['mlp_b2'][l],
                       pf['ln_g'][l], pf['ln_b'][l], 1000)

    dg, deltas = _final(hw, frag_batch.astype(jnp.int32), cond, params)
    return dg, deltas.reshape(-1)


# speedup vs baseline: 3.8581x; 1.3248x over previous
import functools

import jax
import jax.numpy as jnp
from jax import lax
from jax.experimental import pallas as pl
from jax.experimental.pallas import tpu as pltpu
from jax.experimental.pallas import tpu_sc as plsc


def _linear_body(x_ref, w_ref, b_ref, o_ref):
    o_ref[...] = (
        jnp.dot(x_ref[...], w_ref[...], preferred_element_type=jnp.float32)
        + b_ref[...]
    )


def _linear(x, w, b, rb):
    n, din = x.shape
    dout = w.shape[1]
    return pl.pallas_call(
        _linear_body,
        grid=(n // rb,),
        in_specs=[
            pl.BlockSpec((rb, din), lambda i: (i, 0)),
            pl.BlockSpec((din, dout), lambda i: (0, 0)),
            pl.BlockSpec((1, dout), lambda i: (0, 0)),
        ],
        out_specs=pl.BlockSpec((rb, dout), lambda i: (i, 0)),
        out_shape=jax.ShapeDtypeStruct((n, dout), jnp.float32),
    )(x, w, b.reshape(1, -1))


def _edge_embed(ea, w, b, e_pad, eb):
    de, h = w.shape
    return pl.pallas_call(
        _linear_body,
        grid=(e_pad // eb,),
        in_specs=[
            pl.BlockSpec((eb, de), lambda i: (i, 0)),
            pl.BlockSpec((de, h), lambda i: (0, 0)),
            pl.BlockSpec((1, h), lambda i: (0, 0)),
        ],
        out_specs=pl.BlockSpec((eb, h), lambda i: (i, 0)),
        out_shape=jax.ShapeDtypeStruct((e_pad, h), jnp.float32),
    )(ea, w, b.reshape(1, -1))


def _gine_mlp_body(eps_ref, x_ref, agg_ref, w1_ref, b1_ref, w2_ref, b2_ref,
                   g_ref, bb_ref, o_ref):
    eps = eps_ref[0]
    dd = x_ref.shape[1]
    agg = agg_ref[0][:, :dd] + agg_ref[1][:, :dd]
    h = (1.0 + eps) * x_ref[...] + agg
    t = jnp.maximum(
        jnp.dot(h, w1_ref[...], preferred_element_type=jnp.float32)
        + b1_ref[...], 0.0)
    o = (jnp.dot(t, w2_ref[...], preferred_element_type=jnp.float32)
         + b2_ref[...])
    mu = jnp.mean(o, axis=-1, keepdims=True)
    var = jnp.mean(o * o, axis=-1, keepdims=True) - mu * mu
    o = (o - mu) * lax.rsqrt(var + 1e-5) * g_ref[...] + bb_ref[...]
    o_ref[...] = jnp.maximum(o, 0.0)


def _gine_mlp(xx, agg2, eps, w1, b1, w2, b2, g, bb, rb):
    n, dd = xx.shape
    da = agg2.shape[2]
    d2 = w1.shape[1]
    return pl.pallas_call(
        _gine_mlp_body,
        grid=(n // rb,),
        in_specs=[
            pl.BlockSpec(memory_space=pltpu.SMEM),
            pl.BlockSpec((rb, dd), lambda i: (i, 0)),
            pl.BlockSpec((2, rb, da), lambda i: (0, i, 0)),
            pl.BlockSpec((dd, d2), lambda i: (0, 0)),
            pl.BlockSpec((1, d2), lambda i: (0, 0)),
            pl.BlockSpec((d2, dd), lambda i: (0, 0)),
            pl.BlockSpec((1, dd), lambda i: (0, 0)),
            pl.BlockSpec((1, dd), lambda i: (0, 0)),
            pl.BlockSpec((1, dd), lambda i: (0, 0)),
        ],
        out_specs=pl.BlockSpec((rb, dd), lambda i: (i, 0)),
        out_shape=jax.ShapeDtypeStruct((n, dd), jnp.float32),
    )(eps.reshape(1), xx, agg2, w1, b1.reshape(1, -1), w2,
      b2.reshape(1, -1), g.reshape(1, -1), bb.reshape(1, -1))


def _pool_body(a2f_ref, h_ref, w1_ref, b1_ref, w2_ref, b2_ref, o_ref):
    fb, _ = o_ref.shape
    nfull = h_ref.shape[0]
    f0 = pl.program_id(0) * fb
    fid = lax.broadcasted_iota(jnp.int32, (fb, nfull), 0) + f0
    oh = (fid == a2f_ref[...]).astype(jnp.float32)
    sums = jnp.dot(oh, h_ref[...], preferred_element_type=jnp.float32)
    cnt = jnp.sum(oh, axis=1, keepdims=True)
    m = sums / jnp.maximum(cnt, 1.0)
    t = (jnp.dot(m, w1_ref[...], preferred_element_type=jnp.float32)
         + b1_ref[...])
    o_ref[...] = (jnp.dot(t, w2_ref[...], preferred_element_type=jnp.float32)
                  + b2_ref[...])


def _pool(a2f, h_atom, w1, b1, w2, b2, nf, fb):
    n, dd = h_atom.shape
    hf = w1.shape[1]
    return pl.pallas_call(
        _pool_body,
        grid=(nf // fb,),
        in_specs=[
            pl.BlockSpec((1, n), lambda i: (0, 0)),
            pl.BlockSpec((n, dd), lambda i: (0, 0)),
            pl.BlockSpec((dd, hf), lambda i: (0, 0)),
            pl.BlockSpec((1, hf), lambda i: (0, 0)),
            pl.BlockSpec((hf, hf), lambda i: (0, 0)),
            pl.BlockSpec((1, hf), lambda i: (0, 0)),
        ],
        out_specs=pl.BlockSpec((fb, hf), lambda i: (i, 0)),
        out_shape=jax.ShapeDtypeStruct((nf, hf), jnp.float32),
    )(a2f.reshape(1, -1), h_atom, w1, b1.reshape(1, -1), w2,
      b2.reshape(1, -1))


def _final_body(hf_ref, fb_ref, cond_ref, fow1_ref, fob1_ref, fow2_ref,
                fob2_ref, cpw1_ref, cpb1_ref, cpw2_ref, cpb2_ref, gmw1_ref,
                gmb1_ref, gmw2_ref, gmb2_ref, dg_ref, dl_ref):
    hf = hf_ref[...]
    nb = dg_ref.shape[0]
    nf = hf.shape[0]
    hm = hf_ref.shape[1]
    t = jnp.maximum(
        jnp.dot(hf, fow1_ref[...], preferred_element_type=jnp.float32)
        + fob1_ref[...], 0.0)
    deltas = (jnp.dot(t, fow2_ref[...], preferred_element_type=jnp.float32)
              + fob2_ref[...])
    dl_ref[...] = deltas
    oh = (lax.broadcasted_iota(jnp.int32, (nb, nf), 0)
          == fb_ref[...]).astype(jnp.float32)
    fs = jnp.dot(oh, deltas, preferred_element_type=jnp.float32)
    cnt = jnp.sum(oh, axis=1, keepdims=True)
    hmol = (jnp.dot(oh, hf, preferred_element_type=jnp.float32)
            / jnp.maximum(cnt, 1.0))
    hc = jnp.maximum(
        jnp.dot(cond_ref[...], cpw1_ref[...],
                preferred_element_type=jnp.float32) + cpb1_ref[...], 0.0)
    hc = (jnp.dot(hc, cpw2_ref[...], preferred_element_type=jnp.float32)
          + cpb2_ref[...])
    gw = gmw1_ref[...]
    z = (jnp.dot(hmol, gw[:hm, :], preferred_element_type=jnp.float32)
         + jnp.dot(hc, gw[hm:, :], preferred_element_type=jnp.float32)
         + gmb1_ref[...])
    g = (jnp.dot(jnp.maximum(z, 0.0), gmw2_ref[...],
                 preferred_element_type=jnp.float32) + gmb2_ref[...])
    dg_ref[...] = fs + g


def _final(h_frag, frag_batch, cond, p):
    nf, hm = h_frag.shape
    nb = cond.shape[0]
    return pl.pallas_call(
        _final_body,
        out_shape=[
            jax.ShapeDtypeStruct((nb, 1), jnp.float32),
            jax.ShapeDtypeStruct((nf, 1), jnp.float32),
        ],
    )(h_frag, frag_batch.reshape(1, -1), cond,
      p['fo_W1'], p['fo_b1'].reshape(1, -1),
      p['fo_W2'], p['fo_b2'].reshape(1, -1),
      p['cp_W1'], p['cp_b1'].reshape(1, -1),
      p['cp_W2'], p['cp_b2'].reshape(1, -1),
      p['gm_W1'], p['gm_b1'].reshape(1, -1),
      p['gm_W2'], p['gm_b2'].reshape(1, -1))


_ZR = 8


def _sc_edge_pass(x, ea, src3d, dst3d, e_pad, n_acc, ch, kslab):
    d = x.shape[1]
    epw = e_pad // 32
    chunks = epw // ch
    nslabs = chunks // kslab
    rpt = n_acc // 16
    zcopies = rpt // _ZR
    dpe = d // 16

    mesh = plsc.VectorSubcoreMesh(core_axis_name="c", subcore_axis_name="s")

    @functools.partial(
        pl.kernel,
        mesh=mesh,
        out_type=jax.ShapeDtypeStruct((2, n_acc, d), jnp.float32),
        scratch_types=[
            pltpu.VMEM((kslab, ch), jnp.int32),
            pltpu.VMEM((kslab, ch), jnp.int32),
            pltpu.VMEM((ch, d), jnp.float32),
            pltpu.VMEM((ch, d), jnp.float32),
            pltpu.VMEM((ch, d), jnp.float32),
            pltpu.VMEM((ch, d), jnp.float32),
            pltpu.VMEM((_ZR, d), jnp.float32),
            pltpu.VMEM_SHARED((n_acc, d), jnp.float32),
            pltpu.SemaphoreType.DMA,
            pltpu.SemaphoreType.DMA,
            pltpu.SemaphoreType.DMA,
            pltpu.SemaphoreType.DMA,
        ],
    )
    def k(x_hbm, ea_hbm, src_hbm, dst_hbm, out_hbm, sslab, dslab,
          rowsA, msgA, rowsB, msgB, zbuf, acc, semA, semB, ssemA, ssemB):
        c = lax.axis_index("c")
        s = lax.axis_index("s")

        def zrow(r, carry):
            for j in range(dpe):
                zbuf[r, pl.ds(j * 16, 16)] = jnp.zeros((16,), jnp.float32)
            return carry
        lax.fori_loop(0, _ZR, zrow, 0)

        def zcp(kk, carry):
            pltpu.sync_copy(zbuf, acc.at[pl.ds(s * rpt + kk * _ZR, _ZR)])
            return carry
        lax.fori_loop(0, zcopies, zcp, 0)
        plsc.subcore_barrier()

        chunk0 = c * (e_pad // (2 * ch)) + s * chunks
        bufs = [(rowsA, msgA, semA, ssemA), (rowsB, msgB, semB, ssemB)]

        nslabs_ = chunks // kslab

        def slab(si, carry):
            row0 = chunk0 + si * kslab
            slab_id = (c * 16 + s) * nslabs_ + si
            pltpu.sync_copy(src_hbm.at[slab_id], sslab)
            pltpu.sync_copy(dst_hbm.at[slab_id], dslab)

            def issue(j, rows, msg, sem):
                off = (row0 + j) * ch
                pltpu.async_copy(ea_hbm.at[pl.ds(off, ch)], msg, sem)
                pltpu.async_copy(x_hbm.at[sslab.at[j]], rows, sem)

            issue(0, rowsA, msgA, semA)
            for j in range(kslab):
                rows, msg, sem, ssem = bufs[j % 2]
                _, omsg, _, ossem = bufs[1 - j % 2]
                off = (row0 + j) * ch
                pltpu.make_async_copy(ea_hbm.at[pl.ds(off, ch)], msg,
                                      sem).wait()
                pltpu.make_async_copy(x_hbm.at[sslab.at[j]], rows,
                                      sem).wait()

                def rrow(r, cc):
                    for jj in range(dpe):
                        sl = pl.ds(jj * 16, 16)
                        msg[r, sl] = jnp.maximum(msg[r, sl] + rows[r, sl],
                                                 0.0)
                    return cc
                lax.fori_loop(0, ch, rrow, 0)

                if j >= 1:
                    pltpu.make_async_copy(omsg, acc.at[dslab.at[j - 1]],
                                          ossem).wait()
                pltpu.async_copy(msg, acc.at[dslab.at[j]], ssem, add=True)
                if j + 1 < kslab:
                    issue(j + 1, *bufs[1 - j % 2][:3])

            lrows, lmsg, lsem, lssem = bufs[(kslab - 1) % 2]
            pltpu.make_async_copy(lmsg, acc.at[dslab.at[kslab - 1]],
                                  lssem).wait()
            return carry
        lax.fori_loop(0, nslabs, slab, 0)
        plsc.subcore_barrier()
        pltpu.sync_copy(acc.at[pl.ds(s * rpt, rpt)],
                        out_hbm.at[c, pl.ds(s * rpt, rpt)])

    return k(x, ea, src3d, dst3d)


def _pick_kslab(chunks, cap=25):
    for kk in range(min(chunks, cap), 0, -1):
        if chunks % kk == 0:
            return kk
    return 1


def _ceil_to(v, m):
    return -(-v // m) * m


def kernel(x, edge_attr, frag_ea, cond, params, edge_index, atom_to_frag,
           frag_ei, frag_batch):
    pa = params['atom']
    pf = params['frag']
    N, _ = x.shape
    E = edge_attr.shape[0]
    H = pa['proj_W'].shape[1]
    F = frag_batch.shape[0]
    EF = frag_ea.shape[0]
    HF = pf['proj_W'].shape[1]
    LA = pa['edge_W'].shape[0]
    LF = pf['edge_W'].shape[0]

    ch_a = 80
    ch_f = 64
    e_pad = _ceil_to(E, 32 * ch_a)
    n_acc = _ceil_to(N + 1, 16 * _ZR)
    ef_pad = _ceil_to(EF, 32 * ch_f)
    f_acc = _ceil_to(F + 1, 16 * _ZR)
    ks_a = _pick_kslab((e_pad // 32) // ch_a)
    ks_f = _pick_kslab((ef_pad // 32) // ch_f)

    srcA = jnp.concatenate(
        [edge_index[0].astype(jnp.int32),
         jnp.zeros((e_pad - E,), jnp.int32)]).reshape(-1, ks_a, ch_a)
    dstA = jnp.concatenate(
        [edge_index[1].astype(jnp.int32),
         jnp.full((e_pad - E,), N, jnp.int32)]).reshape(-1, ks_a, ch_a)
    srcF = jnp.concatenate(
        [frag_ei[0].astype(jnp.int32),
         jnp.zeros((ef_pad - EF,), jnp.int32)]).reshape(-1, ks_f, ch_f)
    dstF = jnp.concatenate(
        [frag_ei[1].astype(jnp.int32),
         jnp.full((ef_pad - EF,), F, jnp.int32)]).reshape(-1, ks_f, ch_f)

    h = _linear(x, pa['proj_W'], pa['proj_b'], 1000)
    for l in range(LA):
        ea_l = _edge_embed(edge_attr, pa['edge_W'][l], pa['edge_b'][l],
                           e_pad, 4000)
        agg2 = _sc_edge_pass(h, ea_l, srcA, dstA, e_pad, n_acc, ch_a, ks_a)
        h = _gine_mlp(h, agg2, pa['eps'][l], pa['mlp_W1'][l], pa['mlp_b1'][l],
                      pa['mlp_W2'][l], pa['mlp_b2'][l], pa['ln_g'][l],
                      pa['ln_b'][l], 1000)

    hw = _pool(atom_to_frag.astype(jnp.int32), h,
               params['frag_proj_W'], params['frag_proj_b'],
               pf['proj_W'], pf['proj_b'], F, 400)

    for l in range(LF):
        ew_pad = jnp.pad(pf['edge_W'][l], ((0, 0), (0, H - HF)))
        eb_pad = jnp.pad(pf['edge_b'][l], ((0, H - HF),))
        eaf_l = _edge_embed(frag_ea, ew_pad, eb_pad, ef_pad, 4096)
        hwp = jnp.pad(hw, ((0, 0), (0, H - HF)))
        agg2 = _sc_edge_pass(hwp, eaf_l, srcF, dstF, ef_pad, f_acc,
                             ch_f, ks_f)
        hw = _gine_mlp(hw, agg2, pf['eps'][l], pf['mlp_W1'][l],
                       pf['mlp_b1'][l], pf['mlp_W2'][l], pf['mlp_b2'][l],
                       pf['ln_g'][l], pf['ln_b'][l], 1000)

    dg, deltas = _final(hw, frag_batch.astype(jnp.int32), cond, params)
    return dg, deltas.reshape(-1)
